# baseline (speedup 1.0000x reference)
import functools
import math

import jax
import jax.numpy as jnp
from jax import lax
from jax.experimental import pallas as pl
from jax.experimental.pallas import tpu as pltpu

_LANE = 128
_SUBLANE = 8
_CHUNK = 16
_BN = 4


def _dice_kernel(x_ref, y_ref, o_ref, *, b_n, chunk, n_chunks):
    def body(t, carry):
        acc_i, acc_u = carry
        off = pl.multiple_of(t * chunk, chunk)
        xs = x_ref[:, pl.ds(off, chunk), :].astype(jnp.float32)
        ys = y_ref[:, pl.ds(off, chunk), :].astype(jnp.float32)
        s = 0.5 * jnp.tanh(0.5 * xs) + 0.5
        return acc_i + s * ys, acc_u + (s + ys)

    zero = jnp.zeros((b_n, chunk, _LANE), jnp.float32)
    acc_i, acc_u = lax.fori_loop(0, n_chunks, body, (zero, zero))

    i_sl = jnp.sum(acc_i, axis=1, keepdims=True)
    u_sl = jnp.sum(acc_u, axis=1, keepdims=True)
    i_s = jnp.sum(i_sl, axis=2, keepdims=True)
    u_s = jnp.sum(u_sl, axis=2, keepdims=True)
    dc = (2.0 * i_s + 1.0) / (u_s + 1.0)
    o_ref[...] = jnp.broadcast_to(dc[:, 0, :], (b_n, _LANE))[None]


@jax.jit
def kernel(x, y):
    n = x.shape[0]
    d = math.prod(x.shape[1:])
    x2 = x.reshape(n, d)
    y2 = y.reshape(n, d)

    d_tile = _CHUNK * _LANE
    d_pad = pl.cdiv(d, d_tile) * d_tile
    if d_pad != d:
        x2 = jnp.pad(x2, ((0, 0), (0, d_pad - d)), constant_values=-1e9)
        y2 = jnp.pad(y2, ((0, 0), (0, d_pad - d)), constant_values=0)
    rows = d_pad // _LANE

    b_n = _BN if n % _BN == 0 else 1
    num_i = n // b_n

    x3 = x2.reshape(n, rows, _LANE)
    y3 = y2.reshape(n, rows, _LANE)

    kern = functools.partial(
        _dice_kernel, b_n=b_n, chunk=_CHUNK, n_chunks=rows // _CHUNK)

    bytes_in = x3.size * x3.dtype.itemsize + y3.size * y3.dtype.itemsize
    cost = pl.CostEstimate(
        flops=int(8 * n * d_pad),
        transcendentals=int(n * d_pad),
        bytes_accessed=int(bytes_in + n * _LANE * 4),
    )

    dc = pl.pallas_call(
        kern,
        out_shape=jax.ShapeDtypeStruct((num_i, b_n, _LANE), jnp.float32),
        grid=(num_i,),
        in_specs=[
            pl.BlockSpec((b_n, rows, _LANE), lambda i: (i, 0, 0)),
            pl.BlockSpec((b_n, rows, _LANE), lambda i: (i, 0, 0)),
        ],
        out_specs=pl.BlockSpec((1, b_n, _LANE), lambda i: (i, 0, 0)),
        compiler_params=pltpu.CompilerParams(
            dimension_semantics=("parallel",),
            vmem_limit_bytes=48 * 1024 * 1024,
        ),
        cost_estimate=cost,
    )(x3, y3)

    return 1.0 - jnp.mean(dc[:, :, 0])


# speedup vs baseline: 2.5881x; 2.5881x over previous
import functools
import math

import jax
import jax.numpy as jnp
from jax import lax
from jax.experimental import pallas as pl
from jax.experimental.pallas import tpu as pltpu

_LANE = 128
_SUBLANE = 8
_BN = 4


def _dice_kernel(x_ref, y_ref, o_ref, *, b_n, chunk, n_chunks, width):
    def body(t, carry):
        acc_i, acc_u = carry
        off = pl.multiple_of(t * chunk, chunk)
        xs = x_ref[:, pl.ds(off, chunk), :].astype(jnp.float32)
        ys = y_ref[:, pl.ds(off, chunk), :].astype(jnp.float32)
        s = 0.5 * jnp.tanh(0.5 * xs) + 0.5
        return acc_i + s * ys, acc_u + (s + ys)

    zero = jnp.zeros((b_n, chunk, width), jnp.float32)
    acc_i, acc_u = lax.fori_loop(0, n_chunks, body, (zero, zero))

    i_sl = jnp.sum(acc_i, axis=1, keepdims=True)
    u_sl = jnp.sum(acc_u, axis=1, keepdims=True)
    i_s = jnp.sum(i_sl, axis=2, keepdims=True)
    u_s = jnp.sum(u_sl, axis=2, keepdims=True)
    dc = (2.0 * i_s + 1.0) / (u_s + 1.0)
    o_ref[...] = jnp.broadcast_to(dc[:, 0, :], (b_n, _LANE))[None]


def _dice_mean(x3, y3):
    n, rows, width = x3.shape

    b_n = _BN if n % _BN == 0 else 1
    num_i = n // b_n

    chunk = _SUBLANE
    while b_n * chunk * width < 8 * _SUBLANE * _LANE and rows % (2 * chunk) == 0:
        chunk *= 2

    kern = functools.partial(
        _dice_kernel, b_n=b_n, chunk=chunk, n_chunks=rows // chunk,
        width=width)

    bytes_in = x3.size * x3.dtype.itemsize + y3.size * y3.dtype.itemsize
    cost = pl.CostEstimate(
        flops=int(8 * x3.size),
        transcendentals=int(x3.size),
        bytes_accessed=int(bytes_in + num_i * b_n * _LANE * 4),
    )

    dc = pl.pallas_call(
        kern,
        out_shape=jax.ShapeDtypeStruct((num_i, b_n, _LANE), jnp.float32),
        grid=(num_i,),
        in_specs=[
            pl.BlockSpec((b_n, rows, width), lambda i: (i, 0, 0)),
            pl.BlockSpec((b_n, rows, width), lambda i: (i, 0, 0)),
        ],
        out_specs=pl.BlockSpec((1, b_n, _LANE), lambda i: (i, 0, 0)),
        compiler_params=pltpu.CompilerParams(
            dimension_semantics=("parallel",),
            vmem_limit_bytes=48 * 1024 * 1024,
        ),
        cost_estimate=cost,
    )(x3, y3)

    return jnp.mean(dc[:, :, 0])


@jax.jit
def kernel(x, y):
    n = x.shape[0]
    w = x.shape[-1]
    lead = math.prod(x.shape[1:-1])

    if w % _LANE == 0 and lead % _SUBLANE == 0:
        x3 = x.reshape(n, lead, w)
        y3 = y.reshape(n, lead, w)
    else:
        d = lead * w
        d_tile = _SUBLANE * _LANE
        d_pad = pl.cdiv(d, d_tile) * d_tile
        x2 = x.reshape(n, d)
        y2 = y.reshape(n, d)
        if d_pad != d:
            x2 = jnp.pad(x2, ((0, 0), (0, d_pad - d)), constant_values=-1e9)
            y2 = jnp.pad(y2, ((0, 0), (0, d_pad - d)), constant_values=0)
        x3 = x2.reshape(n, d_pad // _LANE, _LANE)
        y3 = y2.reshape(n, d_pad // _LANE, _LANE)

    return 1.0 - _dice_mean(x3, y3)


# speedup vs baseline: 3.3759x; 1.3044x over previous
import functools
import math

import jax
import jax.numpy as jnp
from jax import lax
from jax.experimental import pallas as pl
from jax.experimental.pallas import tpu as pltpu

_LANE = 128
_SUBLANE = 8
_BN = 8


def _dice_kernel(x_ref, y_ref, o_ref, *, b_n, chunk, n_chunks, width):
    def body(t, carry):
        acc_i, acc_u = carry
        off = pl.multiple_of(t * chunk, chunk)
        xs = x_ref[:, pl.ds(off, chunk), :].astype(jnp.float32)
        ys = y_ref[:, pl.ds(off, chunk), :].astype(jnp.float32)
        s = 0.5 * jnp.tanh(0.5 * xs) + 0.5
        return acc_i + s * ys, acc_u + (s + ys)

    zero = jnp.zeros((b_n, chunk, width), jnp.float32)
    acc_i, acc_u = lax.fori_loop(0, n_chunks, body, (zero, zero))

    i_sl = jnp.sum(acc_i, axis=1, keepdims=True)
    u_sl = jnp.sum(acc_u, axis=1, keepdims=True)
    i_s = jnp.sum(i_sl, axis=2, keepdims=True)
    u_s = jnp.sum(u_sl, axis=2, keepdims=True)
    dc = (2.0 * i_s + 1.0) / (u_s + 1.0)
    o_ref[...] = jnp.broadcast_to(dc[:, 0, :], (b_n, _LANE))[None]


def _dice_mean(x3, y3):
    n, rows, width = x3.shape

    b_n = _BN if n % _BN == 0 else 1
    num_i = n // b_n

    chunk = _SUBLANE
    while b_n * chunk * width < 8 * _SUBLANE * _LANE and rows % (2 * chunk) == 0:
        chunk *= 2

    kern = functools.partial(
        _dice_kernel, b_n=b_n, chunk=chunk, n_chunks=rows // chunk,
        width=width)

    bytes_in = x3.size * x3.dtype.itemsize + y3.size * y3.dtype.itemsize
    cost = pl.CostEstimate(
        flops=int(8 * x3.size),
        transcendentals=int(x3.size),
        bytes_accessed=int(bytes_in + num_i * b_n * _LANE * 4),
    )

    dc = pl.pallas_call(
        kern,
        out_shape=jax.ShapeDtypeStruct((num_i, b_n, _LANE), jnp.float32),
        grid=(num_i,),
        in_specs=[
            pl.BlockSpec((b_n, rows, width), lambda i: (i, 0, 0)),
            pl.BlockSpec((b_n, rows, width), lambda i: (i, 0, 0)),
        ],
        out_specs=pl.BlockSpec((1, b_n, _LANE), lambda i: (i, 0, 0)),
        compiler_params=pltpu.CompilerParams(
            dimension_semantics=("parallel",),
            vmem_limit_bytes=48 * 1024 * 1024,
        ),
        cost_estimate=cost,
    )(x3, y3)

    return jnp.mean(dc[:, :, 0])


@jax.jit
def kernel(x, y):
    n = x.shape[0]
    w = x.shape[-1]
    lead = math.prod(x.shape[1:-1])

    if w % _LANE == 0 and lead % _SUBLANE == 0:
        x3 = x.reshape(n, lead, w)
        y3 = y.reshape(n, lead, w)
    else:
        d = lead * w
        d_tile = _SUBLANE * _LANE
        d_pad = pl.cdiv(d, d_tile) * d_tile
        x2 = x.reshape(n, d)
        y2 = y.reshape(n, d)
        if d_pad != d:
            x2 = jnp.pad(x2, ((0, 0), (0, d_pad - d)), constant_values=-1e9)
            y2 = jnp.pad(y2, ((0, 0), (0, d_pad - d)), constant_values=0)
        x3 = x2.reshape(n, d_pad // _LANE, _LANE)
        y3 = y2.reshape(n, d_pad // _LANE, _LANE)

    return 1.0 - _dice_mean(x3, y3)


# speedup vs baseline: 3.5911x; 1.0637x over previous
import functools
import math

import jax
import jax.numpy as jnp
from jax import lax
from jax.experimental import pallas as pl
from jax.experimental.pallas import tpu as pltpu

_LANE = 128
_SUBLANE = 8
_BN = 16


def _dice_kernel(x_ref, y_ref, o_ref, *, b_n, chunk, n_chunks, width):
    def body(t, carry):
        acc_i, acc_u = carry
        off = pl.multiple_of(t * chunk, chunk)
        xs = x_ref[:, pl.ds(off, chunk), :].astype(jnp.float32)
        ys = y_ref[:, pl.ds(off, chunk), :].astype(jnp.float32)
        s = 0.5 * jnp.tanh(0.5 * xs) + 0.5
        return acc_i + s * ys, acc_u + (s + ys)

    zero = jnp.zeros((b_n, chunk, width), jnp.float32)
    acc_i, acc_u = lax.fori_loop(0, n_chunks, body, (zero, zero))

    i_sl = jnp.sum(acc_i, axis=1, keepdims=True)
    u_sl = jnp.sum(acc_u, axis=1, keepdims=True)
    i_s = jnp.sum(i_sl, axis=2, keepdims=True)
    u_s = jnp.sum(u_sl, axis=2, keepdims=True)
    dc = (2.0 * i_s + 1.0) / (u_s + 1.0)
    o_ref[...] = jnp.broadcast_to(dc[:, 0, :], (b_n, _LANE))[None]


def _dice_mean(x3, y3):
    n, rows, width = x3.shape

    b_n = _BN if n % _BN == 0 else 1
    num_i = n // b_n

    chunk = _SUBLANE
    while b_n * chunk * width < 8 * _SUBLANE * _LANE and rows % (2 * chunk) == 0:
        chunk *= 2

    kern = functools.partial(
        _dice_kernel, b_n=b_n, chunk=chunk, n_chunks=rows // chunk,
        width=width)

    bytes_in = x3.size * x3.dtype.itemsize + y3.size * y3.dtype.itemsize
    cost = pl.CostEstimate(
        flops=int(8 * x3.size),
        transcendentals=int(x3.size),
        bytes_accessed=int(bytes_in + num_i * b_n * _LANE * 4),
    )

    dc = pl.pallas_call(
        kern,
        out_shape=jax.ShapeDtypeStruct((num_i, b_n, _LANE), jnp.float32),
        grid=(num_i,),
        in_specs=[
            pl.BlockSpec((b_n, rows, width), lambda i: (i, 0, 0)),
            pl.BlockSpec((b_n, rows, width), lambda i: (i, 0, 0)),
        ],
        out_specs=pl.BlockSpec((1, b_n, _LANE), lambda i: (i, 0, 0)),
        compiler_params=pltpu.CompilerParams(
            dimension_semantics=("parallel",),
            vmem_limit_bytes=48 * 1024 * 1024,
        ),
        cost_estimate=cost,
    )(x3, y3)

    return jnp.mean(dc[:, :, 0])


@jax.jit
def kernel(x, y):
    n = x.shape[0]
    w = x.shape[-1]
    lead = math.prod(x.shape[1:-1])

    if w % _LANE == 0 and lead % _SUBLANE == 0:
        x3 = x.reshape(n, lead, w)
        y3 = y.reshape(n, lead, w)
    else:
        d = lead * w
        d_tile = _SUBLANE * _LANE
        d_pad = pl.cdiv(d, d_tile) * d_tile
        x2 = x.reshape(n, d)
        y2 = y.reshape(n, d)
        if d_pad != d:
            x2 = jnp.pad(x2, ((0, 0), (0, d_pad - d)), constant_values=-1e9)
            y2 = jnp.pad(y2, ((0, 0), (0, d_pad - d)), constant_values=0)
        x3 = x2.reshape(n, d_pad // _LANE, _LANE)
        y3 = y2.reshape(n, d_pad // _LANE, _LANE)

    return 1.0 - _dice_mean(x3, y3)
